# Initial kernel scaffold; baseline (speedup 1.0000x reference)
import jax
import jax.numpy as jnp
from jax.experimental import pallas as pl


def kernel(x, edge_index, batch, W1, a1_src, a1_dst, b1, W2, a2_src, a2_dst, b2, gW1, gb1, bn_gamma, bn_beta, gW2, gb2, glW, glb):
    raise NotImplementedError("write your pallas kernel here")


# speedup vs baseline: 17.7983x; 17.7983x over previous
import functools

import jax
import jax.numpy as jnp
from jax import lax
from jax.experimental import pallas as pl
from jax.experimental.pallas import tpu as pltpu
from jax.experimental.pallas import tpu_sc as plsc

N = 10000
E = 320000
D = 128
H = 4
G = 16

TILES = 16
NCORES = 2
NPAD = 10240
NSC = 10016
ROWW = 144
CH = 128
EPAD = TILES * 162 * CH
PER_TILE = EPAD // TILES
NCHUNK = PER_TILE // CH
NROWS_T = NSC // TILES
BLK = 256


def _edge_pass(srcpad, dstpad, haug_flat, alps, alpd, zrows):
    mesh = plsc.VectorSubcoreMesh(core_axis_name="c", subcore_axis_name="s")

    @functools.partial(
        pl.kernel,
        out_type=jax.ShapeDtypeStruct((H, NPAD, ROWW), jnp.float32),
        mesh=mesh,
        compiler_params=pltpu.CompilerParams(
            needs_layout_passes=False, use_tc_tiling_on_sc=False),
        scratch_types=[
            pltpu.VMEM_SHARED((NSC, ROWW), jnp.float32),
            pltpu.VMEM((NPAD,), jnp.float32),
            pltpu.VMEM((NPAD,), jnp.float32),
            pltpu.VMEM((CH,), jnp.int32),
            pltpu.VMEM((CH,), jnp.int32),
            pltpu.VMEM((CH,), jnp.int32),
            pltpu.VMEM((CH,), jnp.float32),
            pltpu.VMEM((CH, ROWW), jnp.float32),
            pltpu.SemaphoreType.DMA,
        ],
    )
    def kern(src_hbm, dst_hbm, haug_hbm, alps_hbm, alpd_hbm, z_hbm, out_hbm,
             acc_sh, als_v, ald_v, si_v, di_v, sg_v, w_v, rows_v, sem):
        c = lax.axis_index("c")
        s = lax.axis_index("s")
        ebase = s * PER_TILE
        for hh in range(H // NCORES):
            head = c * (H // NCORES) + hh
            pltpu.sync_copy(z_hbm, acc_sh.at[pl.ds(s * NROWS_T, NROWS_T)])
            pltpu.sync_copy(alps_hbm.at[head], als_v)
            pltpu.sync_copy(alpd_hbm.at[head], ald_v)
            plsc.subcore_barrier()

            def chunk(k, carry):
                base = ebase + k * CH
                pltpu.sync_copy(src_hbm.at[pl.ds(base, CH)], si_v)
                pltpu.sync_copy(dst_hbm.at[pl.ds(base, CH)], di_v)
                hbase = head * NPAD
                for i in range(CH // 16):
                    sv = si_v[pl.ds(i * 16, 16)]
                    sg_v[pl.ds(i * 16, 16)] = sv + hbase
                gat = pltpu.async_copy(haug_hbm.at[sg_v], rows_v, sem)
                for i in range(CH // 16):
                    sv = si_v[pl.ds(i * 16, 16)]
                    dv = di_v[pl.ds(i * 16, 16)]
                    a = plsc.load_gather(als_v, [sv]) + plsc.load_gather(ald_v, [dv])
                    e = jnp.where(a >= 0.0, a, 0.2 * a)
                    w_v[pl.ds(i * 16, 16)] = jnp.exp(e)
                gat.wait()

                def scale(g, c2):
                    wv = w_v[pl.ds(g * 16, 16)]
                    for l in range(16):
                        wi = wv[l]
                        i = g * 16 + l
                        for j in range(ROWW // 16):
                            rows_v[i, pl.ds(j * 16, 16)] = (
                                rows_v[i, pl.ds(j * 16, 16)] * wi)
                    return c2

                lax.fori_loop(0, CH // 16, scale, 0)
                pltpu.sync_copy(rows_v, acc_sh.at[di_v], add=True)
                return carry

            lax.fori_loop(0, NCHUNK, chunk, 0)
            plsc.subcore_barrier()
            pltpu.sync_copy(acc_sh.at[pl.ds(s * NROWS_T, NROWS_T)],
                            out_hbm.at[head].at[pl.ds(s * NROWS_T, NROWS_T)])
        plsc.subcore_barrier()

    return kern(srcpad, dstpad, haug_flat, alps, alpd, zrows)


def _prep_body(first, x_or_acc_ref, w_ref, as_ref, ad_ref, b_ref,
               haug_ref, alps_ref, alpd_ref):
    i = pl.program_id(0)
    if first:
        x2 = x_or_acc_ref[...]
    else:
        num = x_or_acc_ref[:, :, 0:D]
        den = x_or_acc_ref[:, :, D:D + 1]
        xm = jnp.sum(num / den, axis=0) * (1.0 / H) + b_ref[...]
        x2 = jax.nn.sigmoid(xm)
        rows = i * BLK + lax.broadcasted_iota(jnp.int32, (BLK, 1), 0)
        x2 = jnp.where(rows < N, x2, 0.0)
    hh = jnp.dot(x2, w_ref[...], preferred_element_type=jnp.float32)
    ones = jnp.ones((BLK, 1), jnp.float32)
    zer = jnp.zeros((BLK, ROWW - D - 1), jnp.float32)
    haug_ref[0] = jnp.concatenate([hh, ones, zer], axis=1)
    alps_ref[0, 0] = jnp.sum(hh * as_ref[0], axis=1)
    alpd_ref[0, 0] = jnp.sum(hh * ad_ref[0], axis=1)


def _prep(first, x_or_acc, w, a_src, a_dst, b):
    grid = (NPAD // BLK, H)
    if first:
        x_spec = pl.BlockSpec((BLK, D), lambda i, h: (i, 0))
    else:
        x_spec = pl.BlockSpec((H, BLK, ROWW), lambda i, h: (0, i, 0))
    return pl.pallas_call(
        functools.partial(_prep_body, first),
        grid=grid,
        in_specs=[
            x_spec,
            pl.BlockSpec((D, D), lambda i, h: (0, h)),
            pl.BlockSpec((1, 1, D), lambda i, h: (h, 0, 0)),
            pl.BlockSpec((1, 1, D), lambda i, h: (h, 0, 0)),
            pl.BlockSpec((1, D), lambda i, h: (0, 0)),
        ],
        out_specs=[
            pl.BlockSpec((1, BLK, ROWW), lambda i, h: (h, i, 0)),
            pl.BlockSpec((1, 1, BLK), lambda i, h: (h, 0, i)),
            pl.BlockSpec((1, 1, BLK), lambda i, h: (h, 0, i)),
        ],
        out_shape=[
            jax.ShapeDtypeStruct((H, NPAD, ROWW), jnp.float32),
            jax.ShapeDtypeStruct((H, 1, NPAD), jnp.float32),
            jax.ShapeDtypeStruct((H, 1, NPAD), jnp.float32),
        ],
    )(x_or_acc, w, a_src.reshape(H, 1, D), a_dst.reshape(H, 1, D), b)


def _x_from_acc_body(acc_ref, b_ref, x_ref):
    i = pl.program_id(0)
    num = acc_ref[:, :, 0:D]
    den = acc_ref[:, :, D:D + 1]
    xm = jnp.sum(num / den, axis=0) * (1.0 / H) + b_ref[...]
    x2 = jax.nn.sigmoid(xm)
    rows = i * BLK + lax.broadcasted_iota(jnp.int32, (BLK, 1), 0)
    x_ref[...] = jnp.where(rows < N, x2, 0.0)


def _x_from_acc(acc, b):
    return pl.pallas_call(
        _x_from_acc_body,
        grid=(NPAD // BLK,),
        in_specs=[
            pl.BlockSpec((H, BLK, ROWW), lambda i: (0, i, 0)),
            pl.BlockSpec((1, D), lambda i: (0, 0)),
        ],
        out_specs=pl.BlockSpec((BLK, D), lambda i: (i, 0)),
        out_shape=jax.ShapeDtypeStruct((NPAD, D), jnp.float32),
    )(acc, b)


def _pool_body(x_ref, batch_ref, gW1_ref, gb1_ref, gam_ref, bet_ref,
               gW2_ref, gb2_ref, glW_ref, glb_ref, out_ref):
    x3 = x_ref[...]
    bt = batch_ref[...]
    mask = (bt < G).astype(jnp.float32)
    g = jnp.dot(x3, gW1_ref[...], preferred_element_type=jnp.float32) + gb1_ref[...]
    mu = jnp.sum(g * mask, axis=0, keepdims=True) * (1.0 / N)
    var = jnp.sum(((g - mu) ** 2) * mask, axis=0, keepdims=True) * (1.0 / N)
    gn = (g - mu) / jnp.sqrt(var + 1e-5) * gam_ref[...] + bet_ref[...]
    gn = jnp.maximum(gn, 0.0)
    gate = jnp.dot(gn, gW2_ref[...], preferred_element_type=jnp.float32) + gb2_ref[...]
    seg = lax.broadcasted_iota(jnp.int32, (1, G), 1)
    onehot = (bt == seg).astype(jnp.float32)
    gateb = jnp.where(bt == seg, gate, -3.0e38)
    gmax = jnp.max(gateb, axis=0, keepdims=True)
    gmaxn = jnp.sum(onehot * gmax, axis=1, keepdims=True)
    ex = jnp.exp(gate - gmaxn) * mask
    deng = jnp.sum(ex * onehot, axis=0, keepdims=True)
    dengn = jnp.sum(onehot * deng, axis=1, keepdims=True)
    attn = ex / (dengn + 1e-16)
    pooled = lax.dot_general(onehot, attn * x3,
                             (((0,), (0,)), ((), ())),
                             preferred_element_type=jnp.float32)
    res = jnp.dot(pooled, glW_ref[...], preferred_element_type=jnp.float32)
    out_ref[...] = jax.nn.sigmoid(res + glb_ref[...])


def _pool(x3, batchpad, gW1, gb1, gamma, beta, gW2, gb2, glW, glb):
    full = lambda shp: pl.BlockSpec(shp, lambda: tuple(0 for _ in shp))
    return pl.pallas_call(
        _pool_body,
        in_specs=[
            full((NPAD, D)), full((NPAD, 1)), full((D, D)), full((1, D)),
            full((1, D)), full((1, D)), full((D, 1)), full((1, 1)),
            full((D, 1)), full((1, 1)),
        ],
        out_specs=full((G, 1)),
        out_shape=jax.ShapeDtypeStruct((G, 1), jnp.float32),
    )(x3, batchpad, gW1, gb1, gamma, beta, gW2, gb2, glW, glb)


def kernel(x, edge_index, batch, W1, a1_src, a1_dst, b1, W2, a2_src, a2_dst, b2,
           gW1, gb1, bn_gamma, bn_beta, gW2, gb2, glW, glb):
    loop = jnp.arange(N, dtype=edge_index.dtype)
    fill = jnp.full((EPAD - E - N,), N, dtype=edge_index.dtype)
    srcpad = jnp.concatenate([edge_index[0], loop, fill])
    dstpad = jnp.concatenate([edge_index[1], loop, fill])
    xpad = jnp.pad(x, ((0, NPAD - N), (0, 0)))
    batchpad = jnp.pad(batch, (0, NPAD - N), constant_values=G).reshape(NPAD, 1)
    zrows = jnp.zeros((NROWS_T, ROWW), jnp.float32)
    b1r = b1.reshape(1, D)
    b2r = b2.reshape(1, D)
    gb1r = gb1.reshape(1, D)
    gamr = bn_gamma.reshape(1, D)
    betr = bn_beta.reshape(1, D)
    gb2r = gb2.reshape(1, 1)
    glbr = glb.reshape(1, 1)

    haug1, alps1, alpd1 = _prep(True, xpad, W1, a1_src, a1_dst, b1r)
    acc1 = _edge_pass(srcpad, dstpad, haug1.reshape(H * NPAD, ROWW),
                      alps1.reshape(H, NPAD), alpd1.reshape(H, NPAD), zrows)
    haug2, alps2, alpd2 = _prep(False, acc1, W2, a2_src, a2_dst, b1r)
    acc2 = _edge_pass(srcpad, dstpad, haug2.reshape(H * NPAD, ROWW),
                      alps2.reshape(H, NPAD), alpd2.reshape(H, NPAD), zrows)
    x3 = _x_from_acc(acc2, b2r)
    return _pool(x3, batchpad, gW1, gb1r, gamr, betr, gW2, gb2r, glW, glbr)


# speedup vs baseline: 20.2649x; 1.1386x over previous
import functools

import jax
import jax.numpy as jnp
from jax import lax
from jax.experimental import pallas as pl
from jax.experimental.pallas import tpu as pltpu
from jax.experimental.pallas import tpu_sc as plsc

N = 10000
E = 320000
D = 128
H = 4
G = 16

TILES = 16
NCORES = 2
NPAD = 10240
NSC = 10016
ROWW = 144
CH = 64
EPAD = TILES * 324 * CH
PER_TILE = EPAD // TILES
NCHUNK = PER_TILE // CH
NROWS_T = NSC // TILES
BLK = 256


def _edge_pass(srcpad, dstpad, haug_flat, alps, alpd, zrows):
    mesh = plsc.VectorSubcoreMesh(core_axis_name="c", subcore_axis_name="s")

    @functools.partial(
        pl.kernel,
        out_type=jax.ShapeDtypeStruct((H, NPAD, ROWW), jnp.float32),
        mesh=mesh,
        compiler_params=pltpu.CompilerParams(
            needs_layout_passes=False, use_tc_tiling_on_sc=False),
        scratch_types=[
            pltpu.VMEM_SHARED((NSC, ROWW), jnp.float32),
            pltpu.VMEM((NPAD,), jnp.float32),
            pltpu.VMEM((NPAD,), jnp.float32),
            [pltpu.VMEM((CH,), jnp.int32)] * 2,
            [pltpu.VMEM((CH,), jnp.int32)] * 2,
            [pltpu.VMEM((CH,), jnp.int32)] * 2,
            [pltpu.VMEM((CH,), jnp.float32)] * 2,
            [pltpu.VMEM((CH, ROWW), jnp.float32)] * 2,
            [pltpu.SemaphoreType.DMA] * 2,
        ],
    )
    def kern(src_hbm, dst_hbm, haug_hbm, alps_hbm, alpd_hbm, z_hbm, out_hbm,
             acc_sh, als_v, ald_v, si_v, di_v, sg_v, w_v, rows_v, sem):
        c = lax.axis_index("c")
        s = lax.axis_index("s")
        ebase = s * PER_TILE
        for hh in range(H // NCORES):
            head = c * (H // NCORES) + hh
            hbase = head * NPAD
            pltpu.sync_copy(z_hbm, acc_sh.at[pl.ds(s * NROWS_T, NROWS_T)])
            pltpu.sync_copy(alps_hbm.at[head], als_v)
            pltpu.sync_copy(alpd_hbm.at[head], ald_v)
            plsc.subcore_barrier()

            def prefetch(k, b):
                base = ebase + k * CH
                pltpu.sync_copy(src_hbm.at[pl.ds(base, CH)], si_v[b])
                pltpu.sync_copy(dst_hbm.at[pl.ds(base, CH)], di_v[b])
                for i in range(CH // 16):
                    sv = si_v[b][pl.ds(i * 16, 16)]
                    dv = di_v[b][pl.ds(i * 16, 16)]
                    sg_v[b][pl.ds(i * 16, 16)] = sv + hbase
                    a = plsc.load_gather(als_v, [sv]) + plsc.load_gather(ald_v, [dv])
                    e = jnp.where(a >= 0.0, a, 0.2 * a)
                    w_v[b][pl.ds(i * 16, 16)] = jnp.exp(e)
                pltpu.async_copy(haug_hbm.at[sg_v[b]], rows_v[b], sem[b])

            def consume(b):
                pltpu.make_async_copy(haug_hbm.at[sg_v[b]], rows_v[b],
                                      sem[b]).wait()

                def scale(g, c2):
                    wv = w_v[b][pl.ds(g * 16, 16)]
                    for l in range(16):
                        wi = wv[l]
                        i = g * 16 + l
                        for j in range(ROWW // 16):
                            rows_v[b][i, pl.ds(j * 16, 16)] = (
                                rows_v[b][i, pl.ds(j * 16, 16)] * wi)
                    return c2

                lax.fori_loop(0, CH // 16, scale, 0)
                pltpu.sync_copy(rows_v[b], acc_sh.at[di_v[b]], add=True)

            prefetch(0, 0)

            def pair(k2, carry):
                prefetch(2 * k2 + 1, 1)
                consume(0)

                @pl.when(k2 < NCHUNK // 2 - 1)
                def _():
                    prefetch(2 * k2 + 2, 0)

                consume(1)
                return carry

            lax.fori_loop(0, NCHUNK // 2, pair, 0)
            plsc.subcore_barrier()
            pltpu.sync_copy(acc_sh.at[pl.ds(s * NROWS_T, NROWS_T)],
                            out_hbm.at[head].at[pl.ds(s * NROWS_T, NROWS_T)])
        plsc.subcore_barrier()

    return kern(srcpad, dstpad, haug_flat, alps, alpd, zrows)


def _prep_body(first, x_or_acc_ref, w_ref, as_ref, ad_ref, b_ref,
               haug_ref, alps_ref, alpd_ref):
    i = pl.program_id(0)
    if first:
        x2 = x_or_acc_ref[...]
    else:
        num = x_or_acc_ref[:, :, 0:D]
        den = x_or_acc_ref[:, :, D:D + 1]
        xm = jnp.sum(num / den, axis=0) * (1.0 / H) + b_ref[...]
        x2 = jax.nn.sigmoid(xm)
        rows = i * BLK + lax.broadcasted_iota(jnp.int32, (BLK, 1), 0)
        x2 = jnp.where(rows < N, x2, 0.0)
    hh = jnp.dot(x2, w_ref[...], preferred_element_type=jnp.float32)
    ones = jnp.ones((BLK, 1), jnp.float32)
    zer = jnp.zeros((BLK, ROWW - D - 1), jnp.float32)
    haug_ref[0] = jnp.concatenate([hh, ones, zer], axis=1)
    alps_ref[0, 0] = jnp.sum(hh * as_ref[0], axis=1)
    alpd_ref[0, 0] = jnp.sum(hh * ad_ref[0], axis=1)


def _prep(first, x_or_acc, w, a_src, a_dst, b):
    grid = (NPAD // BLK, H)
    if first:
        x_spec = pl.BlockSpec((BLK, D), lambda i, h: (i, 0))
    else:
        x_spec = pl.BlockSpec((H, BLK, ROWW), lambda i, h: (0, i, 0))
    return pl.pallas_call(
        functools.partial(_prep_body, first),
        grid=grid,
        in_specs=[
            x_spec,
            pl.BlockSpec((D, D), lambda i, h: (0, h)),
            pl.BlockSpec((1, 1, D), lambda i, h: (h, 0, 0)),
            pl.BlockSpec((1, 1, D), lambda i, h: (h, 0, 0)),
            pl.BlockSpec((1, D), lambda i, h: (0, 0)),
        ],
        out_specs=[
            pl.BlockSpec((1, BLK, ROWW), lambda i, h: (h, i, 0)),
            pl.BlockSpec((1, 1, BLK), lambda i, h: (h, 0, i)),
            pl.BlockSpec((1, 1, BLK), lambda i, h: (h, 0, i)),
        ],
        out_shape=[
            jax.ShapeDtypeStruct((H, NPAD, ROWW), jnp.float32),
            jax.ShapeDtypeStruct((H, 1, NPAD), jnp.float32),
            jax.ShapeDtypeStruct((H, 1, NPAD), jnp.float32),
        ],
    )(x_or_acc, w, a_src.reshape(H, 1, D), a_dst.reshape(H, 1, D), b)


def _x_from_acc_body(acc_ref, b_ref, x_ref):
    i = pl.program_id(0)
    num = acc_ref[:, :, 0:D]
    den = acc_ref[:, :, D:D + 1]
    xm = jnp.sum(num / den, axis=0) * (1.0 / H) + b_ref[...]
    x2 = jax.nn.sigmoid(xm)
    rows = i * BLK + lax.broadcasted_iota(jnp.int32, (BLK, 1), 0)
    x_ref[...] = jnp.where(rows < N, x2, 0.0)


def _x_from_acc(acc, b):
    return pl.pallas_call(
        _x_from_acc_body,
        grid=(NPAD // BLK,),
        in_specs=[
            pl.BlockSpec((H, BLK, ROWW), lambda i: (0, i, 0)),
            pl.BlockSpec((1, D), lambda i: (0, 0)),
        ],
        out_specs=pl.BlockSpec((BLK, D), lambda i: (i, 0)),
        out_shape=jax.ShapeDtypeStruct((NPAD, D), jnp.float32),
    )(acc, b)


def _pool_body(x_ref, batch_ref, gW1_ref, gb1_ref, gam_ref, bet_ref,
               gW2_ref, gb2_ref, glW_ref, glb_ref, out_ref):
    x3 = x_ref[...]
    bt = batch_ref[...]
    mask = (bt < G).astype(jnp.float32)
    g = jnp.dot(x3, gW1_ref[...], preferred_element_type=jnp.float32) + gb1_ref[...]
    mu = jnp.sum(g * mask, axis=0, keepdims=True) * (1.0 / N)
    var = jnp.sum(((g - mu) ** 2) * mask, axis=0, keepdims=True) * (1.0 / N)
    gn = (g - mu) / jnp.sqrt(var + 1e-5) * gam_ref[...] + bet_ref[...]
    gn = jnp.maximum(gn, 0.0)
    gate = jnp.dot(gn, gW2_ref[...], preferred_element_type=jnp.float32) + gb2_ref[...]
    seg = lax.broadcasted_iota(jnp.int32, (1, G), 1)
    onehot = (bt == seg).astype(jnp.float32)
    gateb = jnp.where(bt == seg, gate, -3.0e38)
    gmax = jnp.max(gateb, axis=0, keepdims=True)
    gmaxn = jnp.sum(onehot * gmax, axis=1, keepdims=True)
    ex = jnp.exp(gate - gmaxn) * mask
    deng = jnp.sum(ex * onehot, axis=0, keepdims=True)
    dengn = jnp.sum(onehot * deng, axis=1, keepdims=True)
    attn = ex / (dengn + 1e-16)
    pooled = lax.dot_general(onehot, attn * x3,
                             (((0,), (0,)), ((), ())),
                             preferred_element_type=jnp.float32)
    res = jnp.dot(pooled, glW_ref[...], preferred_element_type=jnp.float32)
    out_ref[...] = jax.nn.sigmoid(res + glb_ref[...])


def _pool(x3, batchpad, gW1, gb1, gamma, beta, gW2, gb2, glW, glb):
    full = lambda shp: pl.BlockSpec(shp, lambda: tuple(0 for _ in shp))
    return pl.pallas_call(
        _pool_body,
        in_specs=[
            full((NPAD, D)), full((NPAD, 1)), full((D, D)), full((1, D)),
            full((1, D)), full((1, D)), full((D, 1)), full((1, 1)),
            full((D, 1)), full((1, 1)),
        ],
        out_specs=full((G, 1)),
        out_shape=jax.ShapeDtypeStruct((G, 1), jnp.float32),
    )(x3, batchpad, gW1, gb1, gamma, beta, gW2, gb2, glW, glb)


def kernel(x, edge_index, batch, W1, a1_src, a1_dst, b1, W2, a2_src, a2_dst, b2,
           gW1, gb1, bn_gamma, bn_beta, gW2, gb2, glW, glb):
    loop = jnp.arange(N, dtype=edge_index.dtype)
    fill = jnp.full((EPAD - E - N,), N, dtype=edge_index.dtype)
    srcpad = jnp.concatenate([edge_index[0], loop, fill])
    dstpad = jnp.concatenate([edge_index[1], loop, fill])
    xpad = jnp.pad(x, ((0, NPAD - N), (0, 0)))
    batchpad = jnp.pad(batch, (0, NPAD - N), constant_values=G).reshape(NPAD, 1)
    zrows = jnp.zeros((NROWS_T, ROWW), jnp.float32)
    b1r = b1.reshape(1, D)
    b2r = b2.reshape(1, D)
    gb1r = gb1.reshape(1, D)
    gamr = bn_gamma.reshape(1, D)
    betr = bn_beta.reshape(1, D)
    gb2r = gb2.reshape(1, 1)
    glbr = glb.reshape(1, 1)

    haug1, alps1, alpd1 = _prep(True, xpad, W1, a1_src, a1_dst, b1r)
    acc1 = _edge_pass(srcpad, dstpad, haug1.reshape(H * NPAD, ROWW),
                      alps1.reshape(H, NPAD), alpd1.reshape(H, NPAD), zrows)
    haug2, alps2, alpd2 = _prep(False, acc1, W2, a2_src, a2_dst, b1r)
    acc2 = _edge_pass(srcpad, dstpad, haug2.reshape(H * NPAD, ROWW),
                      alps2.reshape(H, NPAD), alpd2.reshape(H, NPAD), zrows)
    x3 = _x_from_acc(acc2, b2r)
    return _pool(x3, batchpad, gW1, gb1r, gamr, betr, gW2, gb2r, glW, glbr)
